# Initial kernel scaffold; baseline (speedup 1.0000x reference)
import jax
import jax.numpy as jnp
from jax.experimental import pallas as pl


def kernel(mem, val, idx, retrieve_idx):
    raise NotImplementedError("write your pallas kernel here")


# speedup vs baseline: 3.2320x; 3.2320x over previous
import jax
import jax.numpy as jnp
from jax import lax
from jax.experimental import pallas as pl
from jax.experimental.pallas import tpu as pltpu
from jax.experimental.pallas import tpu_sc as plsc

NC, NS, L = 2, 16, 16
NW = NC * NS


def _mesh():
    return plsc.VectorSubcoreMesh(
        core_axis_name="c", subcore_axis_name="s", num_cores=NC, num_subcores=NS
    )


def _params():
    return pltpu.CompilerParams(needs_layout_passes=False)


def _worker_id():
    return lax.axis_index("s") * NC + lax.axis_index("c")


def _make_scatter(M, D, B):
    assert B % (L * NW) == 0 or B % NW == 0
    bpw = B // NW

    def body(val_hbm, idx_hbm, new_mem_ref, idx_v, posof_v, sidx_v, didx_v,
             rows_v, sem):
        wid = _worker_id()
        lid = lax.iota(jnp.int32, L)

        pltpu.sync_copy(idx_hbm, idx_v)

        @pl.loop(0, B // L)
        def _(c):
            c_v = idx_v[pl.ds(c * L, L)]
            pos_v = c * L + lid
            for k in range(L):
                plsc.store_scatter(posof_v, [c_v], pos_v, mask=lid == k)

        base = wid * bpw
        for t in range(bpw // L):
            c_v = idx_v[pl.ds(base + t * L, L)]
            s_v = plsc.load_gather(posof_v, [c_v])
            sidx_v[...] = s_v
            didx_v[...] = c_v
            pltpu.async_copy(val_hbm.at[sidx_v], rows_v, sem).wait()
            pltpu.async_copy(rows_v, new_mem_ref.at[didx_v], sem).wait()

    return pl.kernel(
        body,
        out_type=(),
        mesh=_mesh(),
        scratch_types=[
            pltpu.VMEM((B,), jnp.int32),
            pltpu.VMEM((M,), jnp.int32),
            pltpu.VMEM((L,), jnp.int32),
            pltpu.VMEM((L,), jnp.int32),
            pltpu.VMEM((L, D), jnp.float32),
            pltpu.SemaphoreType.DMA,
        ],
        compiler_params=_params(),
    )


def _make_gather(M, D, R):
    rpw = R // NW
    CH = 16

    def body(new_mem_hbm, ridx_hbm, out_hbm, ridx_v, rows_v, sem):
        wid = _worker_id()
        base = wid * rpw
        pltpu.sync_copy(ridx_hbm.at[pl.ds(base, rpw)], ridx_v)
        for t in range(rpw // CH):
            pltpu.async_copy(
                new_mem_hbm.at[ridx_v.at[pl.ds(t * CH, CH)]], rows_v, sem
            ).wait()
            pltpu.sync_copy(rows_v, out_hbm.at[pl.ds(base + t * CH, CH)])

    return pl.kernel(
        body,
        out_type=jax.ShapeDtypeStruct((R, D), jnp.float32),
        mesh=_mesh(),
        scratch_types=[
            pltpu.VMEM((rpw,), jnp.int32),
            pltpu.VMEM((CH, D), jnp.float32),
            pltpu.SemaphoreType.DMA,
        ],
        compiler_params=_params(),
    )


def kernel(mem, val, idx, retrieve_idx):
    M, D = mem.shape
    B = idx.shape[0]
    R = retrieve_idx.shape[0]

    new_mem_ref = jax.new_ref(mem)
    _make_scatter(M, D, B)(val, idx, new_mem_ref)
    new_mem = jax.freeze(new_mem_ref)
    retrieved = _make_gather(M, D, R)(new_mem, retrieve_idx)
    return new_mem, retrieved
